# baseline (speedup 1.0000x reference)
import jax
import jax.numpy as jnp
from jax.experimental import pallas as pl
from jax.experimental.pallas import tpu as pltpu

BM = 512
HM = BM // 2
HK = 1024
HN = 1024


def _strassen_kernel(x_ref, w_ref, b_ref, o_ref, c_ref):
    i = pl.program_id(0)

    @pl.when(i == 0)
    def _build_combos():
        w00 = w_ref[:HN, :HK]
        w01 = w_ref[:HN, HK:]
        w10 = w_ref[HN:, :HK]
        w11 = w_ref[HN:, HK:]
        c_ref[0] = (w00 + w11).astype(jnp.bfloat16)
        c_ref[1] = w00.astype(jnp.bfloat16)
        c_ref[2] = (w10 - w11).astype(jnp.bfloat16)
        c_ref[3] = (w01 - w00).astype(jnp.bfloat16)
        c_ref[4] = w11.astype(jnp.bfloat16)
        c_ref[5] = (w00 + w10).astype(jnp.bfloat16)
        c_ref[6] = (w01 + w11).astype(jnp.bfloat16)

    def _dot(a, k):
        return jax.lax.dot_general(
            a, c_ref[k],
            dimension_numbers=(((1,), (1,)), ((), ())),
            preferred_element_type=jnp.float32,
        )

    a11 = x_ref[:HM, :HK].astype(jnp.bfloat16)
    a12 = x_ref[:HM, HK:].astype(jnp.bfloat16)
    a21 = x_ref[HM:, :HK].astype(jnp.bfloat16)
    a22 = x_ref[HM:, HK:].astype(jnp.bfloat16)

    b_lo = b_ref[:, :HN]
    b_hi = b_ref[:, HN:]

    m4 = _dot(a22, 3)
    m5 = _dot(a11 + a12, 4)
    m2 = _dot(a21 + a22, 1)
    m3 = _dot(a11, 2)
    o_ref[HM:, :HN] = m2 + m4 + b_lo
    o_ref[:HM, HN:] = m3 + m5 + b_hi
    p45 = m4 - m5
    p32 = m3 - m2
    m1 = _dot(a11 + a22, 0)
    q11 = m1 + p45 + b_lo
    q22 = m1 + p32 + b_hi
    m6 = _dot(a21 - a11, 5)
    o_ref[HM:, HN:] = q22 + m6
    m7 = _dot(a12 - a22, 6)
    o_ref[:HM, :HN] = q11 + m7


def kernel(input, weight, bias):
    n_tokens, in_f = input.shape
    out_f = weight.shape[0]
    b2 = bias.reshape(1, out_f)
    return pl.pallas_call(
        _strassen_kernel,
        grid=(n_tokens // BM,),
        in_specs=[
            pl.BlockSpec((BM, in_f), lambda i: (i, 0)),
            pl.BlockSpec((out_f, in_f), lambda i: (0, 0)),
            pl.BlockSpec((1, out_f), lambda i: (0, 0)),
        ],
        out_specs=pl.BlockSpec((BM, out_f), lambda i: (i, 0)),
        out_shape=jax.ShapeDtypeStruct((n_tokens, out_f), jnp.float32),
        scratch_shapes=[
            pltpu.VMEM((7, HN, HK), jnp.bfloat16),
        ],
        compiler_params=pltpu.CompilerParams(
            dimension_semantics=("arbitrary",),
        ),
    )(input, weight, b2)


# speedup vs baseline: 1.0191x; 1.0191x over previous
import jax
import jax.numpy as jnp
from jax.experimental import pallas as pl
from jax.experimental.pallas import tpu as pltpu

BM = 512
HM = BM // 2
HK = 1024
HN = 1024


def _strassen_kernel(x_ref, w_ref, b_ref, o_ref, c_ref):
    i = pl.program_id(0)

    @pl.when(i == 0)
    def _build_combos():
        w00 = w_ref[:HN, :HK]
        w01 = w_ref[:HN, HK:]
        w10 = w_ref[HN:, :HK]
        w11 = w_ref[HN:, HK:]
        c_ref[0] = (w00 + w11).astype(jnp.bfloat16)
        c_ref[1] = w00.astype(jnp.bfloat16)
        c_ref[2] = (w10 - w11).astype(jnp.bfloat16)
        c_ref[3] = (w01 - w00).astype(jnp.bfloat16)
        c_ref[4] = w11.astype(jnp.bfloat16)
        c_ref[5] = (w00 + w10).astype(jnp.bfloat16)
        c_ref[6] = (w01 + w11).astype(jnp.bfloat16)

    def _dot(a, k):
        return jax.lax.dot_general(
            a, c_ref[k],
            dimension_numbers=(((1,), (1,)), ((), ())),
            preferred_element_type=jnp.float32,
        )

    a11 = x_ref[:HM, :HK].astype(jnp.bfloat16)
    a12 = x_ref[:HM, HK:].astype(jnp.bfloat16)
    a21 = x_ref[HM:, :HK].astype(jnp.bfloat16)
    a22 = x_ref[HM:, HK:].astype(jnp.bfloat16)

    b_lo = b_ref[:, :HN]
    b_hi = b_ref[:, HN:]

    m1 = _dot(a11 + a22, 0)
    m2 = _dot(a21 + a22, 1)
    m3 = _dot(a11, 2)
    m4 = _dot(a22, 3)
    m5 = _dot(a11 + a12, 4)
    q11 = m1 + m4 - m5 + b_lo
    q22 = m1 - m2 + m3 + b_hi
    m6 = _dot(a21 - a11, 5)
    m7 = _dot(a12 - a22, 6)
    o_ref[:HM, :HN] = q11 + m7
    o_ref[:HM, HN:] = m3 + m5 + b_hi
    o_ref[HM:, :HN] = m2 + m4 + b_lo
    o_ref[HM:, HN:] = q22 + m6


def kernel(input, weight, bias):
    n_tokens, in_f = input.shape
    out_f = weight.shape[0]
    b2 = bias.reshape(1, out_f)
    return pl.pallas_call(
        _strassen_kernel,
        grid=(n_tokens // BM,),
        in_specs=[
            pl.BlockSpec((BM, in_f), lambda i: (i, 0)),
            pl.BlockSpec((out_f, in_f), lambda i: (0, 0)),
            pl.BlockSpec((1, out_f), lambda i: (0, 0)),
        ],
        out_specs=pl.BlockSpec((BM, out_f), lambda i: (i, 0)),
        out_shape=jax.ShapeDtypeStruct((n_tokens, out_f), jnp.float32),
        scratch_shapes=[
            pltpu.VMEM((7, HN, HK), jnp.bfloat16),
        ],
        compiler_params=pltpu.CompilerParams(
            dimension_semantics=("arbitrary",),
        ),
    )(input, weight, b2)
